# baseline (speedup 1.0000x reference)
import jax
import jax.numpy as jnp
from jax.experimental import pallas as pl
from jax.experimental.pallas import tpu as pltpu

_C, _H, _W = 3, 4096, 4096
_BH = 128
_G = _H // _BH


def _tv_body(x_ref, o_ref, lr_ref):
    i = pl.program_id(0)

    xa = x_ref[...]
    d = pltpu.roll(xa, _W - 1, 2) - xa
    e = pltpu.roll(xa, _BH - 1, 1) - xa
    tot = (d * d + e * e).reshape(_C * _BH, _W)
    wcol = jnp.where(
        jax.lax.broadcasted_iota(jnp.int32, (_W, 8), 0) < _W - 1, 1.0, 0.0
    ).astype(jnp.float32)
    partial = jax.lax.dot_general(
        tot, wcol, (((1,), (0,)), ((), ())),
        preferred_element_type=jnp.float32)
    s = jnp.sum(partial) * (1.0 / 8.0)

    r0 = x_ref[:, 0, :]
    rL = x_ref[:, _BH - 1, :]
    wr = r0 - rL
    wrv = wr[:, :-1]
    s = s - jnp.sum(wrv * wrv)
    sd = r0 - lr_ref[:, 0, :]
    sdv = sd[:, :-1]
    s = s + jnp.where(i > 0, jnp.sum(sdv * sdv), 0.0)
    lrdx = rL[:, 1:] - rL[:, :-1]
    s = s - jnp.where(i == _G - 1, jnp.sum(lrdx * lrdx), 0.0)

    @pl.when(i == 0)
    def _():
        o_ref[...] = jnp.zeros_like(o_ref)

    o_ref[...] += s
    lr_ref[:, 0, :] = rL

    @pl.when(i == _G - 1)
    def _():
        o_ref[...] = jnp.sqrt(o_ref[...])


def kernel(input):
    x = input.reshape(_C, _H, _W)
    out = pl.pallas_call(
        _tv_body,
        grid=(_G,),
        in_specs=[
            pl.BlockSpec((_C, _BH, _W), lambda i: (0, i, 0)),
        ],
        scratch_shapes=[pltpu.VMEM((_C, 8, _W), jnp.float32)],
        out_specs=pl.BlockSpec((1, 1, 128), lambda i: (0, 0, 0)),
        out_shape=jax.ShapeDtypeStruct((1, 1, 128), jnp.float32),
        compiler_params=pltpu.CompilerParams(
            dimension_semantics=("arbitrary",),
            vmem_limit_bytes=57 * 1024 * 1024,
        ),
        name="tv_loss",
    )(x)
    return out[0, 0, 0]


# speedup vs baseline: 1.0599x; 1.0599x over previous
import jax
import jax.numpy as jnp
from jax.experimental import pallas as pl
from jax.experimental.pallas import tpu as pltpu

_C, _H, _W = 3, 4096, 4096
_BH = 256
_G = _H // _BH


def _tv_body(x_ref, o_ref, lr_ref):
    i = pl.program_id(0)

    xa = x_ref[...]
    d = pltpu.roll(xa, _W - 1, 2) - xa
    e = pltpu.roll(xa, _BH - 1, 1) - xa
    tot = (d * d + e * e).reshape(_C * _BH, _W)
    wcol = jnp.where(
        jax.lax.broadcasted_iota(jnp.int32, (_W, 8), 0) < _W - 1, 1.0, 0.0
    ).astype(jnp.float32)
    partial = jax.lax.dot_general(
        tot, wcol, (((1,), (0,)), ((), ())),
        preferred_element_type=jnp.float32)
    s = jnp.sum(partial) * (1.0 / 8.0)

    r0 = x_ref[:, 0, :]
    rL = x_ref[:, _BH - 1, :]
    wr = r0 - rL
    wrv = wr[:, :-1]
    s = s - jnp.sum(wrv * wrv)
    sd = r0 - lr_ref[:, 0, :]
    sdv = sd[:, :-1]
    s = s + jnp.where(i > 0, jnp.sum(sdv * sdv), 0.0)
    lrdx = rL[:, 1:] - rL[:, :-1]
    s = s - jnp.where(i == _G - 1, jnp.sum(lrdx * lrdx), 0.0)

    @pl.when(i == 0)
    def _():
        o_ref[...] = jnp.zeros_like(o_ref)

    o_ref[...] += s
    lr_ref[:, 0, :] = rL

    @pl.when(i == _G - 1)
    def _():
        o_ref[...] = jnp.sqrt(o_ref[...])


def kernel(input):
    x = input.reshape(_C, _H, _W)
    out = pl.pallas_call(
        _tv_body,
        grid=(_G,),
        in_specs=[
            pl.BlockSpec((_C, _BH, _W), lambda i: (0, i, 0)),
        ],
        scratch_shapes=[pltpu.VMEM((_C, 8, _W), jnp.float32)],
        out_specs=pl.BlockSpec((1, 1, 128), lambda i: (0, 0, 0)),
        out_shape=jax.ShapeDtypeStruct((1, 1, 128), jnp.float32),
        compiler_params=pltpu.CompilerParams(
            dimension_semantics=("arbitrary",),
            vmem_limit_bytes=57 * 1024 * 1024,
        ),
        name="tv_loss",
    )(x)
    return out[0, 0, 0]
